# baseline (speedup 1.0000x reference)
import functools

import jax
import jax.numpy as jnp
from jax import lax
from jax.experimental import pallas as pl
from jax.experimental.pallas import tpu as pltpu
from jax.experimental.pallas import tpu_sc as plsc

B = 1024
L = 20
DIM = 100
ALPHA = 0.2
BB = 8
R = BB * L
DIMP = 128
NW = 32
CHUNK = 128


def _sc_gather(table, idx):
    n = idx.shape[0]
    d = table.shape[1]
    per_w = n // NW
    nchunks = per_w // CHUNK
    mesh = plsc.VectorSubcoreMesh(core_axis_name="c", subcore_axis_name="s")

    @functools.partial(
        pl.kernel,
        out_type=jax.ShapeDtypeStruct((n, d), jnp.float32),
        mesh=mesh,
        scratch_types=[
            pltpu.VMEM((per_w,), jnp.int32),
            pltpu.VMEM((CHUNK, d), jnp.float32),
            pltpu.SemaphoreType.DMA,
        ],
        name="sc_gather3",
    )
    def k(table_hbm, idx_hbm, out_hbm, idx_v, rows_v, sem):
        wid = lax.axis_index("s") * 2 + lax.axis_index("c")
        base = wid * per_w
        pltpu.sync_copy(idx_hbm.at[pl.ds(base, per_w)], idx_v)

        def body(i, carry):
            off = i * CHUNK
            pltpu.async_copy(
                table_hbm.at[idx_v.at[pl.ds(off, CHUNK)]], rows_v, sem
            ).wait()
            pltpu.sync_copy(rows_v, out_hbm.at[pl.ds(base + off, CHUNK)])
            return carry

        lax.fori_loop(0, nchunks, body, 0)

    return k(table, idx)


def _leaky(x):
    return jnp.where(x >= 0, x, ALPHA * x)


def _tc_body(h1_ref, h2_ref, hm_ref, adj_ref, tadj_ref, adjid_ref,
             la1_ref, mix_ref, wei_ref, weo_ref,
             wr_in_ref, wi_in_ref, wn_in_ref, wr_out_ref, wi_out_ref, wn_out_ref,
             whr_ref, whi_ref, whn_ref,
             bei_ref, beo_ref, biah_ref, boah_ref,
             bihr_ref, bihi_ref, bihn_ref, bhhr_ref, bhhi_ref, bhhn_ref,
             o1_ref, o2_ref, om_ref):
    row = lax.broadcasted_iota(jnp.int32, (R, R), 0) // L
    col = lax.broadcasted_iota(jnp.int32, (R, R), 1) // L
    bd = row == col
    big_neg = jnp.float32(-9e15)

    def local_agg(h, adj2d, a_ref):
        adjt = jnp.concatenate([adj2d] * BB, axis=1)
        alpha = jnp.full((R, R), big_neg, jnp.float32)
        for k in range(4):
            a_k = a_ref[k:k + 1, :]
            g = jnp.dot(h * a_k, h.T, preferred_element_type=jnp.float32)
            e_k = _leaky(g)
            alpha = jnp.where(bd & (adjt == k + 1), e_k, alpha)
        alpha = alpha - jnp.max(alpha, axis=1, keepdims=True)
        p = jnp.exp(alpha)
        alpha = p / jnp.sum(p, axis=1, keepdims=True)
        return jnp.dot(alpha, h, preferred_element_type=jnp.float32)

    h1 = h1_ref[:, :DIM]
    h2 = h2_ref[:, :DIM]
    hm = hm_ref[:, :DIM]

    o1_ref[...] = local_agg(h1, adj_ref[...], la1_ref)
    om_ref[...] = local_agg(hm, tadj_ref[...], mix_ref)

    adjid = adjid_ref[...]
    a_in = jnp.where(bd, jnp.concatenate([adjid[:, :L]] * BB, axis=1), 0.0)
    a_out = jnp.where(bd, jnp.concatenate([adjid[:, L:]] * BB, axis=1), 0.0)
    x_in = jnp.dot(h2, wei_ref[...], preferred_element_type=jnp.float32) + bei_ref[...]
    x_out = jnp.dot(h2, weo_ref[...], preferred_element_type=jnp.float32) + beo_ref[...]
    input_in = jnp.dot(a_in, x_in, preferred_element_type=jnp.float32) + biah_ref[...]
    input_out = jnp.dot(a_out, x_out, preferred_element_type=jnp.float32) + boah_ref[...]

    def mm(x, w_ref):
        return jnp.dot(x, w_ref[...], preferred_element_type=jnp.float32)

    gi_r = mm(input_in, wr_in_ref) + mm(input_out, wr_out_ref) + bihr_ref[...]
    gi_i = mm(input_in, wi_in_ref) + mm(input_out, wi_out_ref) + bihi_ref[...]
    gi_n = mm(input_in, wn_in_ref) + mm(input_out, wn_out_ref) + bihn_ref[...]
    gh_r = mm(h2, whr_ref) + bhhr_ref[...]
    gh_i = mm(h2, whi_ref) + bhhi_ref[...]
    gh_n = mm(h2, whn_ref) + bhhn_ref[...]
    resetgate = jax.nn.sigmoid(gi_r + gh_r)
    inputgate = jax.nn.sigmoid(gi_i + gh_i)
    newgate = jnp.tanh(gi_n + resetgate * gh_n)
    o2_ref[...] = newgate + inputgate * (newgate - h2)


def _tc_compute(h1g, h2g, hmg, adj2d, tadj2d, adjid2d, weights):
    grid = B // BB
    hrow_spec = pl.BlockSpec((R, DIMP), lambda g: (g, 0))
    row_spec = pl.BlockSpec((R, DIM), lambda g: (g, 0))
    full = lambda s: pl.BlockSpec(s, lambda g: (0, 0))
    in_specs = [
        hrow_spec, hrow_spec, hrow_spec,
        pl.BlockSpec((R, L), lambda g: (g, 0)),
        pl.BlockSpec((R, L), lambda g: (g, 0)),
        pl.BlockSpec((R, 2 * L), lambda g: (g, 0)),
        full((4, DIM)), full((4, DIM)),
    ] + [full((DIM, DIM))] * 11 + [full((1, DIM))] * 10
    out_specs = (row_spec, row_spec, row_spec)
    out_shape = tuple(jax.ShapeDtypeStruct((B * L, DIM), jnp.float32) for _ in range(3))
    return pl.pallas_call(
        _tc_body,
        grid=grid,
        in_specs=in_specs,
        out_specs=out_specs,
        out_shape=out_shape,
        compiler_params=pltpu.CompilerParams(
            dimension_semantics=("arbitrary",),
        ),
    )(h1g, h2g, hmg, adj2d, tadj2d, adjid2d, *weights)


def kernel(inputs, adj, mask_item, item, items_ID, adj_ID, total_items, total_adj,
           embedding, la1_a, mix_a, Wei, bei, Weo, beo, w_ih, w_hh, b_ih, b_hh,
           b_iah, b_oah):
    n = B * L
    idx_all = jnp.concatenate([
        inputs.reshape(-1), items_ID.reshape(-1), total_items.reshape(-1)
    ]).astype(jnp.int32)
    emb_p = jnp.pad(embedding, ((0, 0), (0, DIMP - DIM)))
    rows = _sc_gather(emb_p, idx_all)
    h1g = rows[:n]
    h2g = rows[n:2 * n]
    hmg = rows[2 * n:]

    r1 = lambda v: v.reshape(1, DIM)
    weights = (
        la1_a.T, mix_a.T,
        Wei.T, Weo.T,
        w_ih[0:DIM, 0:DIM].T, w_ih[DIM:2 * DIM, 0:DIM].T, w_ih[2 * DIM:, 0:DIM].T,
        w_ih[0:DIM, DIM:].T, w_ih[DIM:2 * DIM, DIM:].T, w_ih[2 * DIM:, DIM:].T,
        w_hh[0:DIM].T, w_hh[DIM:2 * DIM].T, w_hh[2 * DIM:].T,
        r1(bei), r1(beo), r1(b_iah), r1(b_oah),
        r1(b_ih[0:DIM]), r1(b_ih[DIM:2 * DIM]), r1(b_ih[2 * DIM:]),
        r1(b_hh[0:DIM]), r1(b_hh[DIM:2 * DIM]), r1(b_hh[2 * DIM:]),
    )
    o1, o2, om = _tc_compute(
        h1g, h2g, hmg,
        adj.reshape(n, L).astype(jnp.int32),
        total_adj.reshape(n, L).astype(jnp.int32),
        adj_ID.reshape(n, 2 * L),
        weights,
    )
    return (o1.reshape(B, L, DIM), o2.reshape(B, L, DIM), om.reshape(B, L, DIM))


# speedup vs baseline: 1.4572x; 1.4572x over previous
import functools

import jax
import jax.numpy as jnp
from jax import lax
from jax.experimental import pallas as pl
from jax.experimental.pallas import tpu as pltpu
from jax.experimental.pallas import tpu_sc as plsc

B = 1024
L = 20
DIM = 100
ALPHA = 0.2
BB = 8
R = BB * L
DIMP = 128
NW = 32
CHUNK = 128


def _sc_gather(table, idx):
    n = idx.shape[0]
    d = table.shape[1]
    per_w = n // NW
    nchunks = per_w // CHUNK
    mesh = plsc.VectorSubcoreMesh(core_axis_name="c", subcore_axis_name="s")

    @functools.partial(
        pl.kernel,
        out_type=jax.ShapeDtypeStruct((n, d), jnp.float32),
        mesh=mesh,
        scratch_types=[
            pltpu.VMEM((per_w,), jnp.int32),
            pltpu.VMEM((CHUNK, d), jnp.float32),
            pltpu.SemaphoreType.DMA,
        ],
        name="sc_gather3",
    )
    def k(table_hbm, idx_hbm, out_hbm, idx_v, rows_v, sem):
        wid = lax.axis_index("s") * 2 + lax.axis_index("c")
        base = wid * per_w
        pltpu.sync_copy(idx_hbm.at[pl.ds(base, per_w)], idx_v)

        def body(i, carry):
            off = i * CHUNK
            pltpu.async_copy(
                table_hbm.at[idx_v.at[pl.ds(off, CHUNK)]], rows_v, sem
            ).wait()
            pltpu.sync_copy(rows_v, out_hbm.at[pl.ds(base + off, CHUNK)])
            return carry

        lax.fori_loop(0, nchunks, body, 0)

    return k(table, idx)


def _pad_rows(emb):
    rows = emb.shape[0]
    blk = 2000

    def body(s_ref, d_ref):
        d_ref[:, :DIM] = s_ref[...]
        d_ref[:, DIM:] = jnp.zeros((blk, DIMP - DIM), jnp.float32)

    return pl.pallas_call(
        body,
        grid=rows // blk,
        in_specs=[pl.BlockSpec((blk, DIM), lambda g: (g, 0))],
        out_specs=pl.BlockSpec((blk, DIMP), lambda g: (g, 0)),
        out_shape=jax.ShapeDtypeStruct((rows, DIMP), jnp.float32),
        compiler_params=pltpu.CompilerParams(
            dimension_semantics=("arbitrary",),
        ),
    )(emb)


def _leaky(x):
    return jnp.where(x >= 0, x, ALPHA * x)


def _tc_body(h1_ref, h2_ref, hm_ref, adj_ref, tadj_ref, adjid_ref,
             la1_ref, mix_ref, wei_ref, weo_ref,
             wr_in_ref, wi_in_ref, wn_in_ref, wr_out_ref, wi_out_ref, wn_out_ref,
             whr_ref, whi_ref, whn_ref,
             bei_ref, beo_ref, biah_ref, boah_ref,
             bihr_ref, bihi_ref, bihn_ref, bhhr_ref, bhhi_ref, bhhn_ref,
             o1_ref, o2_ref, om_ref):
    row = lax.broadcasted_iota(jnp.int32, (R, R), 0) // L
    col = lax.broadcasted_iota(jnp.int32, (R, R), 1) // L
    bd = row == col
    big_neg = jnp.float32(-9e15)

    def local_agg(h, adj3d, a_ref, o_ref):
        adj2d = adj3d.reshape(R, L)
        adjt = jnp.where(bd, jnp.concatenate([adj2d] * BB, axis=1), 0)
        alpha = jnp.full((R, R), big_neg, jnp.float32)
        for k in range(4):
            a_k = a_ref[k:k + 1, :]
            g = jnp.dot(h * a_k, h.T, preferred_element_type=jnp.float32)
            e_k = _leaky(g)
            alpha = jnp.where(adjt == k + 1, e_k, alpha)
        alpha = alpha - jnp.max(alpha, axis=1, keepdims=True)
        p = jnp.exp(alpha)
        alpha = p / jnp.sum(p, axis=1, keepdims=True)
        o = jnp.dot(alpha, h, preferred_element_type=jnp.float32)
        o_ref[...] = o.reshape(BB, L, DIM)

    h1 = h1_ref[:, :DIM]
    h2 = h2_ref[:, :DIM]
    hm = hm_ref[:, :DIM]

    local_agg(h1, adj_ref[...], la1_ref, o1_ref)
    local_agg(hm, tadj_ref[...], mix_ref, om_ref)

    adjid = adjid_ref[...].reshape(R, 2 * L)
    a_in = jnp.where(bd, jnp.concatenate([adjid[:, :L]] * BB, axis=1), 0.0)
    a_out = jnp.where(bd, jnp.concatenate([adjid[:, L:]] * BB, axis=1), 0.0)
    x_in = jnp.dot(h2, wei_ref[...], preferred_element_type=jnp.float32) + bei_ref[...]
    x_out = jnp.dot(h2, weo_ref[...], preferred_element_type=jnp.float32) + beo_ref[...]
    input_in = jnp.dot(a_in, x_in, preferred_element_type=jnp.float32) + biah_ref[...]
    input_out = jnp.dot(a_out, x_out, preferred_element_type=jnp.float32) + boah_ref[...]

    def mm(x, w_ref):
        return jnp.dot(x, w_ref[...], preferred_element_type=jnp.float32)

    gi_r = mm(input_in, wr_in_ref) + mm(input_out, wr_out_ref) + bihr_ref[...]
    gi_i = mm(input_in, wi_in_ref) + mm(input_out, wi_out_ref) + bihi_ref[...]
    gi_n = mm(input_in, wn_in_ref) + mm(input_out, wn_out_ref) + bihn_ref[...]
    gh_r = mm(h2, whr_ref) + bhhr_ref[...]
    gh_i = mm(h2, whi_ref) + bhhi_ref[...]
    gh_n = mm(h2, whn_ref) + bhhn_ref[...]
    resetgate = jax.nn.sigmoid(gi_r + gh_r)
    inputgate = jax.nn.sigmoid(gi_i + gh_i)
    newgate = jnp.tanh(gi_n + resetgate * gh_n)
    o2_ref[...] = (newgate + inputgate * (newgate - h2)).reshape(BB, L, DIM)


def _tc_compute(rows, adj, tadj, adj_ID, weights):
    grid = B // BB
    nb = (B * L) // R
    full = lambda s: pl.BlockSpec(s, lambda g: (0, 0))
    in_specs = [
        pl.BlockSpec((R, DIMP), lambda g: (g, 0)),
        pl.BlockSpec((R, DIMP), lambda g: (g + nb, 0)),
        pl.BlockSpec((R, DIMP), lambda g: (g + 2 * nb, 0)),
        pl.BlockSpec((BB, L, L), lambda g: (g, 0, 0)),
        pl.BlockSpec((BB, L, L), lambda g: (g, 0, 0)),
        pl.BlockSpec((BB, L, 2 * L), lambda g: (g, 0, 0)),
        full((4, DIM)), full((4, DIM)),
    ] + [full((DIM, DIM))] * 11 + [full((1, DIM))] * 10
    o_spec = pl.BlockSpec((BB, L, DIM), lambda g: (g, 0, 0))
    out_specs = (o_spec, o_spec, o_spec)
    out_shape = tuple(jax.ShapeDtypeStruct((B, L, DIM), jnp.float32) for _ in range(3))
    return pl.pallas_call(
        _tc_body,
        grid=grid,
        in_specs=in_specs,
        out_specs=out_specs,
        out_shape=out_shape,
        compiler_params=pltpu.CompilerParams(
            dimension_semantics=("arbitrary",),
        ),
    )(rows, rows, rows, adj, tadj, adj_ID, *weights)


def kernel(inputs, adj, mask_item, item, items_ID, adj_ID, total_items, total_adj,
           embedding, la1_a, mix_a, Wei, bei, Weo, beo, w_ih, w_hh, b_ih, b_hh,
           b_iah, b_oah):
    n = B * L
    idx_all = jnp.concatenate([
        inputs.reshape(-1), items_ID.reshape(-1), total_items.reshape(-1)
    ]).astype(jnp.int32)
    emb_p = _pad_rows(embedding)
    rows = _sc_gather(emb_p, idx_all)

    r1 = lambda v: v.reshape(1, DIM)
    weights = (
        la1_a.T, mix_a.T,
        Wei.T, Weo.T,
        w_ih[0:DIM, 0:DIM].T, w_ih[DIM:2 * DIM, 0:DIM].T, w_ih[2 * DIM:, 0:DIM].T,
        w_ih[0:DIM, DIM:].T, w_ih[DIM:2 * DIM, DIM:].T, w_ih[2 * DIM:, DIM:].T,
        w_hh[0:DIM].T, w_hh[DIM:2 * DIM].T, w_hh[2 * DIM:].T,
        r1(bei), r1(beo), r1(b_iah), r1(b_oah),
        r1(b_ih[0:DIM]), r1(b_ih[DIM:2 * DIM]), r1(b_ih[2 * DIM:]),
        r1(b_hh[0:DIM]), r1(b_hh[DIM:2 * DIM]), r1(b_hh[2 * DIM:]),
    )
    return _tc_compute(rows, adj.astype(jnp.int32), total_adj.astype(jnp.int32),
                       adj_ID, weights)
